# baseline (speedup 1.0000x reference)
import functools

import jax
import jax.numpy as jnp
from jax import lax
from jax.experimental import pallas as pl
from jax.experimental.pallas import tpu as pltpu
from jax.experimental.pallas import tpu_sc as plsc

B = 4096
N_DENSE = 13
N_SPARSE = 26
NF = N_SPARSE // 2
VOCAB = 100000
VOCAB_PAD = 100096
NCOL = N_DENSE + N_SPARSE
GROUP_STRIDE = 782 * 1024

_info = plsc.get_sparse_core_info()
NC = _info.num_cores
NS = _info.num_subcores
L = _info.num_lanes
NW = NC * NS
BPW = B // NW
NCH = BPW // L

_mesh = plsc.VectorSubcoreMesh(core_axis_name="c", subcore_axis_name="s")

_SCRATCH = [
    pltpu.VMEM((NCOL, BPW), jnp.float32),
    pltpu.VMEM((L,), jnp.float32),
    pltpu.VMEM((NF, BPW), jnp.int32),
    pltpu.VMEM((NF, BPW), jnp.float32),
    pltpu.VMEM((BPW,), jnp.float32),
    pltpu.SemaphoreType.DMA,
]


def _half_body(first, x_hbm, tab_hbm, w_hbm, out_hbm,
               xv, wv, idxv, valsv, outv, sem):
    wid = lax.axis_index("s") * NC + lax.axis_index("c")
    base = wid * BPW
    f0 = 0 if first else NF

    pltpu.sync_copy(x_hbm.at[wid], xv)
    if first:
        pltpu.sync_copy(w_hbm, wv)
    else:
        pltpu.sync_copy(w_hbm.at[pl.ds(base, BPW)], outv)

    def build_f(fl, _):
        row = N_DENSE + f0 + fl
        cf = (fl // 8) * GROUP_STRIDE + (fl % 8) * 128

        def build_c(c, _):
            ids = xv[row, pl.ds(c * L, L)].astype(jnp.int32)
            idxv[fl, pl.ds(c * L, L)] = cf + (ids >> 7) * 1024 + (ids & 127)
            return 0

        lax.fori_loop(0, NCH, build_c, 0, unroll=True)
        pltpu.async_copy(tab_hbm.at[idxv.at[fl]], valsv.at[fl], sem)
        return 0

    lax.fori_loop(0, NF, build_f, 0)

    if first:
        wvec = wv[pl.ds(0, L)]

        def dense_c(c, _):
            sl = pl.ds(c * L, L)
            acc = xv[0, sl] * wvec[0]
            for d in range(1, N_DENSE):
                acc = acc + xv[d, sl] * wvec[d]
            outv[sl] = acc
            return 0

        lax.fori_loop(0, NCH, dense_c, 0)

    def drain(fl, _):
        pltpu.make_async_copy(tab_hbm.at[idxv.at[fl]], valsv.at[fl], sem).wait()
        return 0

    lax.fori_loop(0, NF, drain, 0)

    def red_c(c, _):
        sl = pl.ds(c * L, L)
        acc = outv[sl]

        def red_f(fl, a):
            return a + valsv[fl, sl]

        outv[sl] = lax.fori_loop(0, NF, red_f, acc)
        return 0

    lax.fori_loop(0, NCH, red_c, 0)

    pltpu.sync_copy(outv, out_hbm.at[pl.ds(base, BPW)])


_kernel1 = functools.partial(
    pl.kernel, out_type=jax.ShapeDtypeStruct((B,), jnp.float32),
    mesh=_mesh, scratch_types=_SCRATCH,
)(functools.partial(_half_body, True))

_kernel2 = functools.partial(
    pl.kernel, out_type=jax.ShapeDtypeStruct((B,), jnp.float32),
    mesh=_mesh, scratch_types=_SCRATCH,
)(functools.partial(_half_body, False))


def _tile_flat(rows):
    z = jnp.pad(rows, ((0, 16 - NF), (0, VOCAB_PAD - VOCAB)))
    return z.reshape(2, 8, 782, 128).transpose(0, 2, 1, 3).reshape(-1)


def kernel(X, tables, W):
    xr = X.reshape(NW, BPW, NCOL).transpose(0, 2, 1)
    t2 = tables[:, :, 0]
    th1 = _tile_flat(t2[:NF])
    th2 = _tile_flat(t2[NF:])
    wp = jnp.pad(W.reshape(-1), (0, L - N_DENSE))
    part = _kernel1(xr, th1, wp)
    out = _kernel2(xr, th2, part)
    return out.reshape(B, 1)


# speedup vs baseline: 1.0473x; 1.0473x over previous
import functools

import jax
import jax.numpy as jnp
from jax import lax
from jax.experimental import pallas as pl
from jax.experimental.pallas import tpu as pltpu
from jax.experimental.pallas import tpu_sc as plsc

B = 4096
N_DENSE = 13
N_SPARSE = 26
VOCAB = 100000
VOCAB_PAD = 100096
NCOL = N_DENSE + N_SPARSE

_info = plsc.get_sparse_core_info()
NC = _info.num_cores
NS = _info.num_subcores
L = _info.num_lanes
NW = NC * NS
BPW = B // NW
NCH = BPW // L

_mesh = plsc.VectorSubcoreMesh(core_axis_name="c", subcore_axis_name="s")


@functools.partial(
    pl.kernel,
    out_type=jax.ShapeDtypeStruct((B,), jnp.float32),
    mesh=_mesh,
    scratch_types=[
        pltpu.VMEM((NCOL, BPW), jnp.float32),
        pltpu.VMEM((L,), jnp.float32),
        pltpu.VMEM((N_SPARSE, BPW), jnp.int32),
        pltpu.VMEM((N_SPARSE, BPW), jnp.float32),
        pltpu.VMEM((BPW,), jnp.float32),
        pltpu.SemaphoreType.DMA,
    ],
)
def _sc_linear(x_hbm, tab_hbm, w_hbm, out_hbm, xv, wv, idxv, valsv, outv, sem):
    wid = lax.axis_index("s") * NC + lax.axis_index("c")
    base = wid * BPW

    pltpu.sync_copy(x_hbm.at[wid], xv)
    pltpu.sync_copy(w_hbm, wv)

    def build_f(f, _):
        row = N_DENSE + f

        cf = (f // 8) * (782 * 1024) + (f % 8) * 128

        def build_c(c, _):
            ids = xv[row, pl.ds(c * L, L)].astype(jnp.int32)
            idxv[f, pl.ds(c * L, L)] = cf + (ids >> 7) * 1024 + (ids & 127)
            return 0

        lax.fori_loop(0, NCH, build_c, 0, unroll=True)
        pltpu.async_copy(tab_hbm.at[idxv.at[f]], valsv.at[f], sem)
        return 0

    lax.fori_loop(0, N_SPARSE, build_f, 0)

    wvec = wv[pl.ds(0, L)]

    def dense_c(c, _):
        sl = pl.ds(c * L, L)
        acc = xv[0, sl] * wvec[0]
        for d in range(1, N_DENSE):
            acc = acc + xv[d, sl] * wvec[d]
        outv[sl] = acc
        return 0

    lax.fori_loop(0, NCH, dense_c, 0)

    def drain(f, _):
        pltpu.make_async_copy(tab_hbm.at[idxv.at[f]], valsv.at[f], sem).wait()
        return 0

    lax.fori_loop(0, N_SPARSE, drain, 0)

    def red_c(c, _):
        sl = pl.ds(c * L, L)
        acc = outv[sl]

        def red_f(f, a):
            return a + valsv[f, sl]

        outv[sl] = lax.fori_loop(0, N_SPARSE, red_f, acc)
        return 0

    lax.fori_loop(0, NCH, red_c, 0)

    pltpu.sync_copy(outv, out_hbm.at[pl.ds(base, BPW)])


def kernel(X, tables, W):
    xr = X.reshape(NW, BPW, NCOL).transpose(0, 2, 1)
    z = jnp.pad(tables[:, :, 0], ((0, 6), (0, VOCAB_PAD - VOCAB)))
    tab_flat = z.reshape(4, 8, 782, 128).transpose(0, 2, 1, 3).reshape(-1)
    wp = jnp.pad(W.reshape(-1), (0, L - N_DENSE))
    out = _sc_linear(xr, tab_flat, wp)
    return out.reshape(B, 1)
